# baseline (speedup 1.0000x reference)
import functools

import jax
import jax.numpy as jnp
from jax import lax
from jax.experimental import pallas as pl
from jax.experimental.pallas import tpu as pltpu
from jax.experimental.pallas import tpu_sc as plsc

VOCAB = 100000
D_MODEL = 768
NUM_LABELS = 2
LPAD = 16

_ROW_BLK = 5000

_NC, _NS = 2, 16
_NW = _NC * _NS
_CHUNK = 80


_KSPLIT = 3
_KW = D_MODEL // _KSPLIT


def _proj_body(t0_ref, t1_ref, t2_ref, w_ref, b_ref, o_ref):
    acc = b_ref[...]
    for k, t_ref in enumerate((t0_ref, t1_ref, t2_ref)):
        acc = acc + lax.dot_general(
            t_ref[...], w_ref[:, k * _KW:(k + 1) * _KW],
            dimension_numbers=(((1,), (1,)), ((), ())),
            preferred_element_type=jnp.float32,
            precision=lax.Precision.DEFAULT,
        )
    o_ref[...] = acc


def _project_table(emb_table, probe_w, b2):
    grid = (VOCAB // _ROW_BLK,)
    panel = lambda k: pl.BlockSpec((_ROW_BLK, _KW), lambda i, _k=k: (i, _k))
    return pl.pallas_call(
        _proj_body,
        grid=grid,
        in_specs=[
            panel(0), panel(1), panel(2),
            pl.BlockSpec((LPAD, D_MODEL), lambda i: (0, 0)),
            pl.BlockSpec((1, LPAD), lambda i: (0, 0)),
        ],
        out_specs=pl.BlockSpec((_ROW_BLK, LPAD), lambda i: (i, 0)),
        out_shape=jax.ShapeDtypeStruct((VOCAB, LPAD), jnp.float32),
    )(emb_table, emb_table, emb_table, probe_w, b2)


def _make_gather(bsz, seq):
    n_idx = bsz * seq
    per_w = n_idx // _NW
    n_chunks = per_w // _CHUNK
    seq_f = seq * NUM_LABELS
    mesh = plsc.VectorSubcoreMesh(core_axis_name="c", subcore_axis_name="s")

    @functools.partial(
        pl.kernel,
        mesh=mesh,
        compiler_params=pltpu.CompilerParams(
            use_tc_tiling_on_sc=False, needs_layout_passes=False),
        out_type=jax.ShapeDtypeStruct((bsz, seq, NUM_LABELS), jnp.float32),
        scratch_types=[
            pltpu.VMEM((n_chunks, _CHUNK), jnp.int32),
            pltpu.VMEM((per_w, LPAD), jnp.float32),
            pltpu.VMEM((bsz // _NW, seq, NUM_LABELS), jnp.float32),
            pltpu.SemaphoreType.DMA,
        ],
    )
    def gather_k(idx_hbm, proj_hbm, out_hbm, idx_v, rows_v, out_v, sem):
        wid = lax.axis_index("s") * _NC + lax.axis_index("c")
        pltpu.sync_copy(idx_hbm.at[wid], idx_v)
        copies = []
        for j in range(n_chunks):
            copies.append(pltpu.async_copy(
                proj_hbm.at[idx_v.at[j]],
                rows_v.at[pl.ds(j * _CHUNK, _CHUNK)],
                sem))
        for c in copies:
            c.wait()
        lane = lax.iota(jnp.int32, 16)
        half = lane >> 1
        col = lane & 1
        n_out = per_w * NUM_LABELS
        for j in range(n_out // 16):
            vals = plsc.load_gather(rows_v, [half + (j * 8), col])
            f = j * 16 + lane
            b = f // seq_f
            r = f - b * seq_f
            plsc.store_scatter(out_v, [b, r >> 1, col], vals)
        rows_per_w = bsz // _NW
        pltpu.sync_copy(out_v,
                        out_hbm.at[pl.ds(wid * rows_per_w, rows_per_w)])

    return gather_k


def kernel(sentences, emb_table, probe_w, probe_b):
    bsz, seq = sentences.shape
    n_idx = bsz * seq

    w_pad = jnp.pad(probe_w, ((0, LPAD - NUM_LABELS), (0, 0)))
    b_pad = jnp.pad(probe_b, (0, LPAD - NUM_LABELS)).reshape(1, LPAD)
    proj = _project_table(emb_table, w_pad, b_pad)

    per_w = n_idx // _NW
    idx = sentences.astype(jnp.int32).reshape(_NW, per_w // _CHUNK, _CHUNK)
    return _make_gather(bsz, seq)(idx, proj)


# speedup vs baseline: 1.1454x; 1.1454x over previous
import functools

import jax
import jax.numpy as jnp
from jax import lax
from jax.experimental import pallas as pl
from jax.experimental.pallas import tpu as pltpu
from jax.experimental.pallas import tpu_sc as plsc

VOCAB = 100000
D_MODEL = 768
NUM_LABELS = 2
LPAD = 16

_ROW_BLK = 5000

_NC, _NS = 2, 16
_NW = _NC * _NS
_CHUNK = 80


_KSPLIT = 3
_KW = D_MODEL // _KSPLIT


def _proj_body(t0_ref, t1_ref, t2_ref, w_ref, b_ref, o_ref):
    acc = b_ref[...]
    for k, t_ref in enumerate((t0_ref, t1_ref, t2_ref)):
        acc = acc + lax.dot_general(
            t_ref[...], w_ref[:, k * _KW:(k + 1) * _KW],
            dimension_numbers=(((1,), (1,)), ((), ())),
            preferred_element_type=jnp.float32,
            precision=lax.Precision.DEFAULT,
        )
    o_ref[...] = acc


def _project_table(emb_table, probe_w, b2):
    grid = (VOCAB // _ROW_BLK,)
    panel = lambda k: pl.BlockSpec((_ROW_BLK, _KW), lambda i, _k=k: (i, _k))
    return pl.pallas_call(
        _proj_body,
        grid=grid,
        in_specs=[
            panel(0), panel(1), panel(2),
            pl.BlockSpec((LPAD, D_MODEL), lambda i: (0, 0)),
            pl.BlockSpec((1, LPAD), lambda i: (0, 0)),
        ],
        out_specs=pl.BlockSpec((_ROW_BLK, LPAD), lambda i: (i, 0)),
        out_shape=jax.ShapeDtypeStruct((VOCAB, LPAD), jnp.float32),
    )(emb_table, emb_table, emb_table, probe_w, b2)


def _make_gather(bsz, seq):
    n_idx = bsz * seq
    per_w = n_idx // _NW
    n_chunks = per_w // _CHUNK
    seq_f = seq * NUM_LABELS
    mesh = plsc.VectorSubcoreMesh(core_axis_name="c", subcore_axis_name="s")

    @functools.partial(
        pl.kernel,
        mesh=mesh,
        compiler_params=pltpu.CompilerParams(
            use_tc_tiling_on_sc=False, needs_layout_passes=False),
        out_type=jax.ShapeDtypeStruct((n_idx, LPAD), jnp.float32),
        scratch_types=[
            pltpu.VMEM((n_chunks, _CHUNK), jnp.int32),
            pltpu.VMEM((per_w, LPAD), jnp.float32),
            pltpu.SemaphoreType.DMA,
        ],
    )
    def gather_k(idx_hbm, proj_hbm, out_hbm, idx_v, rows_v, sem):
        wid = lax.axis_index("s") * _NC + lax.axis_index("c")
        pltpu.sync_copy(idx_hbm.at[wid], idx_v)
        copies = []
        for j in range(n_chunks):
            copies.append(pltpu.async_copy(
                proj_hbm.at[idx_v.at[j]],
                rows_v.at[pl.ds(j * _CHUNK, _CHUNK)],
                sem))
        for c in copies:
            c.wait()
        pltpu.sync_copy(rows_v, out_hbm.at[pl.ds(wid * per_w, per_w)])

    return gather_k


def kernel(sentences, emb_table, probe_w, probe_b):
    bsz, seq = sentences.shape
    n_idx = bsz * seq

    w_pad = jnp.pad(probe_w, ((0, LPAD - NUM_LABELS), (0, 0)))
    b_pad = jnp.pad(probe_b, (0, LPAD - NUM_LABELS)).reshape(1, LPAD)
    proj = _project_table(emb_table, w_pad, b_pad)

    per_w = n_idx // _NW
    idx = sentences.astype(jnp.int32).reshape(_NW, per_w // _CHUNK, _CHUNK)
    gathered = _make_gather(bsz, seq)(idx, proj)
    sel = jnp.zeros((LPAD, NUM_LABELS), jnp.float32).at[
        jnp.arange(NUM_LABELS), jnp.arange(NUM_LABELS)].set(1.0)
    return (gathered @ sel).reshape(bsz, seq, NUM_LABELS)
